# baseline (speedup 1.0000x reference)
import functools

import jax
import jax.numpy as jnp
from jax import lax
from jax.experimental import pallas as pl
from jax.experimental.pallas import tpu as pltpu
from jax.experimental.pallas import tpu_sc as plsc

B = 16384
D = 64
NC, NS = 2, 16
NW = NC * NS
BPW = B // NW
CHUNK = 128
NCHUNK = BPW // CHUNK


@functools.cache
def _make_sc_gather():
    mesh = plsc.VectorSubcoreMesh(core_axis_name="c", subcore_axis_name="s")

    @functools.partial(
        pl.kernel,
        out_type=jax.ShapeDtypeStruct((B, 2 * D), jnp.float32),
        mesh=mesh,
        scratch_types=[
            pltpu.VMEM((NCHUNK, CHUNK), jnp.int32),
            pltpu.VMEM((NCHUNK, CHUNK), jnp.int32),
            pltpu.VMEM((BPW, D), jnp.float32),
            pltpu.VMEM((BPW, D), jnp.float32),
            pltpu.SemaphoreType.DMA,
            pltpu.SemaphoreType.DMA,
            pltpu.SemaphoreType.DMA,
            pltpu.SemaphoreType.DMA,
        ],
        compiler_params=pltpu.CompilerParams(use_tc_tiling_on_sc=False),
    )
    def sc_gather(user_tbl, item_tbl, uid, iid, out,
                  uidx, iidx, urows, irows, dsem, usem, isem, wsem):
        wid = lax.axis_index("s") * NC + lax.axis_index("c")
        base = wid * BPW
        idxc = []
        for j in range(NCHUNK):
            idxc.append(pltpu.async_copy(
                uid.at[pl.ds(base + j * CHUNK, CHUNK)], uidx.at[j], dsem))
            idxc.append(pltpu.async_copy(
                iid.at[pl.ds(base + j * CHUNK, CHUNK)], iidx.at[j], dsem))
        for c in idxc:
            c.wait()
        ucopies = []
        icopies = []
        for j in range(NCHUNK):
            dst = pl.ds(j * CHUNK, CHUNK)
            ucopies.append(pltpu.async_copy(user_tbl.at[uidx.at[j]], urows.at[dst], usem))
            icopies.append(pltpu.async_copy(item_tbl.at[iidx.at[j]], irows.at[dst], isem))
        for c in ucopies:
            c.wait()
        rows = pl.ds(base, BPW)
        wu = pltpu.async_copy(urows, out.at[rows, pl.ds(0, D)], wsem)
        for c in icopies:
            c.wait()
        wi = pltpu.async_copy(irows, out.at[rows, pl.ds(D, D)], wsem)
        wu.wait()
        wi.wait()

    return sc_gather


BS = 2048
OROWS = BS // 128


def _mlp_body(x, w1t, b1, w2t, b2, w3t, b3, out):
    xb = x[:].astype(jnp.bfloat16)
    h = lax.dot_general(xb, w1t[:].astype(jnp.bfloat16), (((1,), (1,)), ((), ())),
                        preferred_element_type=jnp.float32)
    h = jnp.maximum(h + b1[:], 0.0).astype(jnp.bfloat16)
    h = lax.dot_general(h, w2t[:].astype(jnp.bfloat16), (((1,), (1,)), ((), ())),
                        preferred_element_type=jnp.float32)
    h = jnp.maximum(h + b2[:], 0.0)
    ot = lax.dot_general(w3t[:], h, (((1,), (1,)), ((), ())),
                         preferred_element_type=jnp.float32) + b3[:]
    for r in range(OROWS):
        out[r:r + 1, :] = ot[:, r * 128:(r + 1) * 128]


_mlp_call = pl.pallas_call(
    _mlp_body,
    grid=(B // BS,),
    in_specs=[
        pl.BlockSpec((BS, 2 * D), lambda i: (i, 0)),
        pl.BlockSpec((D, 2 * D), lambda i: (0, 0)),
        pl.BlockSpec((1, D), lambda i: (0, 0)),
        pl.BlockSpec((32, D), lambda i: (0, 0)),
        pl.BlockSpec((1, 32), lambda i: (0, 0)),
        pl.BlockSpec((1, 32), lambda i: (0, 0)),
        pl.BlockSpec((1, 1), lambda i: (0, 0)),
    ],
    out_specs=pl.BlockSpec((OROWS, 128), lambda i: (i, 0)),
    out_shape=jax.ShapeDtypeStruct((B // 128, 128), jnp.float32),
    compiler_params=pltpu.CompilerParams(dimension_semantics=("parallel",)),
)


def kernel(user_id, item_id, user_table, item_table, W1, b1, W2, b2, W3, b3):
    uid = user_id.astype(jnp.int32)
    iid = item_id.astype(jnp.int32)
    xcat = _make_sc_gather()(user_table, item_table, uid, iid)
    outp = _mlp_call(xcat, W1.T, b1.reshape(1, D), W2.T, b2.reshape(1, 32),
                     W3.reshape(1, 32), b3.reshape(1, 1))
    return outp.reshape(B, 1)


# speedup vs baseline: 1.0658x; 1.0658x over previous
import functools

import jax
import jax.numpy as jnp
from jax import lax
from jax.experimental import pallas as pl
from jax.experimental.pallas import tpu as pltpu
from jax.experimental.pallas import tpu_sc as plsc

B = 16384
D = 64
NC, NS = 2, 16
NW = NC * NS
BPW = B // NW
CHUNK = 128
NCHUNK = BPW // CHUNK


@functools.cache
def _make_sc_gather():
    mesh = plsc.VectorSubcoreMesh(core_axis_name="c", subcore_axis_name="s")

    @functools.partial(
        pl.kernel,
        out_type=jax.ShapeDtypeStruct((B, 2 * D), jnp.float32),
        mesh=mesh,
        scratch_types=[
            pltpu.VMEM((NCHUNK, CHUNK), jnp.int32),
            pltpu.VMEM((NCHUNK, CHUNK), jnp.int32),
            pltpu.VMEM((BPW, D), jnp.float32),
            pltpu.VMEM((BPW, D), jnp.float32),
            pltpu.SemaphoreType.DMA,
            pltpu.SemaphoreType.DMA,
            pltpu.SemaphoreType.DMA,
            pltpu.SemaphoreType.DMA,
        ],
        compiler_params=pltpu.CompilerParams(use_tc_tiling_on_sc=False),
    )
    def sc_gather(user_tbl, item_tbl, uid, iid, out,
                  uidx, iidx, urows, irows, dsem, usem, isem, wsem):
        wid = lax.axis_index("s") * NC + lax.axis_index("c")
        base = wid * BPW
        idxc = []
        for j in range(NCHUNK):
            idxc.append(pltpu.async_copy(
                uid.at[pl.ds(base + j * CHUNK, CHUNK)], uidx.at[j], dsem))
            idxc.append(pltpu.async_copy(
                iid.at[pl.ds(base + j * CHUNK, CHUNK)], iidx.at[j], dsem))
        for c in idxc:
            c.wait()
        ucopies = []
        icopies = []
        for j in range(NCHUNK):
            dst = pl.ds(j * CHUNK, CHUNK)
            ucopies.append(pltpu.async_copy(user_tbl.at[uidx.at[j]], urows.at[dst], usem))
            icopies.append(pltpu.async_copy(item_tbl.at[iidx.at[j]], irows.at[dst], isem))
        for c in ucopies:
            c.wait()
        rows = pl.ds(base, BPW)
        wu = pltpu.async_copy(urows, out.at[rows, pl.ds(0, D)], wsem)
        for c in icopies:
            c.wait()
        wi = pltpu.async_copy(irows, out.at[rows, pl.ds(D, D)], wsem)
        wu.wait()
        wi.wait()

    return sc_gather


BS = 8192
OROWS = BS // 128


def _mlp_body(x, w1t, b1, w2t, b2, w3t, b3, out):
    xb = x[:].astype(jnp.bfloat16)
    h = lax.dot_general(xb, w1t[:].astype(jnp.bfloat16), (((1,), (1,)), ((), ())),
                        preferred_element_type=jnp.float32)
    h = jnp.maximum(h + b1[:], 0.0).astype(jnp.bfloat16)
    h = lax.dot_general(h, w2t[:].astype(jnp.bfloat16), (((1,), (1,)), ((), ())),
                        preferred_element_type=jnp.float32)
    h = jnp.maximum(h + b2[:], 0.0)
    ot = lax.dot_general(w3t[:], h, (((1,), (1,)), ((), ())),
                         preferred_element_type=jnp.float32) + b3[:]
    for r in range(OROWS):
        out[r:r + 1, :] = ot[:, r * 128:(r + 1) * 128]


_mlp_call = pl.pallas_call(
    _mlp_body,
    grid=(B // BS,),
    in_specs=[
        pl.BlockSpec((BS, 2 * D), lambda i: (i, 0)),
        pl.BlockSpec((D, 2 * D), lambda i: (0, 0)),
        pl.BlockSpec((1, D), lambda i: (0, 0)),
        pl.BlockSpec((32, D), lambda i: (0, 0)),
        pl.BlockSpec((1, 32), lambda i: (0, 0)),
        pl.BlockSpec((1, 32), lambda i: (0, 0)),
        pl.BlockSpec((1, 1), lambda i: (0, 0)),
    ],
    out_specs=pl.BlockSpec((OROWS, 128), lambda i: (i, 0)),
    out_shape=jax.ShapeDtypeStruct((B // 128, 128), jnp.float32),
    compiler_params=pltpu.CompilerParams(dimension_semantics=("parallel",)),
)


def kernel(user_id, item_id, user_table, item_table, W1, b1, W2, b2, W3, b3):
    uid = user_id.astype(jnp.int32)
    iid = item_id.astype(jnp.int32)
    xcat = _make_sc_gather()(user_table, item_table, uid, iid)
    outp = _mlp_call(xcat, W1.T, b1.reshape(1, D), W2.T, b2.reshape(1, 32),
                     W3.reshape(1, 32), b3.reshape(1, 1))
    return outp.reshape(B, 1)
